# baseline (speedup 1.0000x reference)
import functools

import jax
import jax.numpy as jnp
from jax import lax
from jax.experimental import pallas as pl
from jax.experimental.pallas import tpu as pltpu
from jax.experimental.pallas import tpu_sc as plsc

ENTITY_N = 100000
RELATION_N = 1000
D = 128
B = 4096
NW = 32
BPW = B // NW

_GATHER_DNUMS = lax.GatherDimensionNumbers(
    offset_dims=(), collapsed_slice_dims=(0,), start_index_map=(0,))


def _lane_bcast(v, i):
    idx = jnp.full((16, 1), i, jnp.int32)
    return lax.gather(v, idx, _GATHER_DNUMS, slice_sizes=(1,),
                      mode=lax.GatherScatterMode.PROMISE_IN_BOUNDS)


def _inv_norm(sv):
    iv = plsc.bitcast(sv, jnp.int32)
    iv = jnp.int32(0x5F3759DF) - lax.shift_right_logical(iv, 1)
    y = plsc.bitcast(iv, jnp.float32)
    y = y * (1.5 - 0.5 * sv * y * y)
    y = y * (1.5 - 0.5 * sv * y * y)
    y = y * (1.5 - 0.5 * sv * y * y)
    n = sv * y
    return 1.0 / jnp.maximum(n, 1e-12)


def _make_sc_kernel():
    mesh = plsc.VectorSubcoreMesh(core_axis_name="c", subcore_axis_name="s")

    @functools.partial(
        pl.kernel,
        out_type=jax.ShapeDtypeStruct((3 * B, D), jnp.float32),
        mesh=mesh,
        compiler_params=pltpu.CompilerParams(needs_layout_passes=False),
        scratch_types=[
            pltpu.VMEM((3 * BPW,), jnp.int32),
            pltpu.VMEM((BPW,), jnp.int32),
            pltpu.VMEM((BPW,), jnp.int32),
            pltpu.VMEM((BPW,), jnp.int32),
            pltpu.VMEM((BPW, D), jnp.float32),
            pltpu.VMEM((BPW, D), jnp.float32),
            pltpu.VMEM((BPW, D), jnp.float32),
            pltpu.VMEM((3 * BPW, D), jnp.float32),
            pltpu.SemaphoreType.DMA,
        ],
    )
    def body(sample_flat, entity, relation, out,
             sblk, ih_v, ir_v, it_v, buf_h, buf_r, buf_t, obuf, sem):
        wid = lax.axis_index("s") * 2 + lax.axis_index("c")
        b0 = wid * BPW
        lanes = lax.iota(jnp.int32, 16)

        pltpu.sync_copy(sample_flat.at[pl.ds(3 * b0, 3 * BPW)], sblk)
        for m in range(BPW // 16):
            row3 = (m * 16 + lanes) * 3
            for c, dst in ((0, ih_v), (1, ir_v), (2, it_v)):
                dst[pl.ds(m * 16, 16)] = plsc.load_gather(sblk, [row3 + c])

        ch = pltpu.async_copy(entity.at[ih_v], buf_h, sem)
        cr = pltpu.async_copy(relation.at[ir_v], buf_r, sem)
        ct = pltpu.async_copy(entity.at[it_v], buf_t, sem)
        ch.wait()
        cr.wait()
        ct.wait()

        def blk_body(blk, _):
            r0 = blk * 16
            rows = r0 + lanes
            for c, buf in ((0, buf_h), (1, buf_r), (2, buf_t)):
                acc = [jnp.zeros((16,), jnp.float32) for _ in range(4)]
                col = jnp.zeros((16,), jnp.int32)
                for j in range(D):
                    v = plsc.load_gather(buf, [rows, col])
                    acc[j % 4] = acc[j % 4] + v * v
                    col = col + 1
                inv = _inv_norm((acc[0] + acc[1]) + (acc[2] + acc[3]))
                for i in range(16):
                    iv = _lane_bcast(inv, i)
                    src = r0 + i
                    dst = 3 * src + c
                    for k in range(D // 16):
                        obuf[dst, pl.ds(16 * k, 16)] = (
                            buf[src, pl.ds(16 * k, 16)] * iv)
            return 0

        lax.fori_loop(0, BPW // 16, blk_body, 0)
        pltpu.sync_copy(obuf, out.at[pl.ds(3 * b0, 3 * BPW)])

    return body


_sc_kernel = _make_sc_kernel()


def kernel(sample, entity_emb, relation_emb, loss_emb):
    del loss_emb
    flat = _sc_kernel(sample.reshape(-1).astype(jnp.int32),
                      entity_emb, relation_emb)
    return flat.reshape(B, 3, D)


# speedup vs baseline: 1.7452x; 1.7452x over previous
import functools

import jax
import jax.numpy as jnp
from jax import lax
from jax.experimental import pallas as pl
from jax.experimental.pallas import tpu as pltpu
from jax.experimental.pallas import tpu_sc as plsc

ENTITY_N = 100000
RELATION_N = 1000
D = 128
B = 4096
NW = 32
BPW = B // NW
BT = 256


def _make_sc_gather():
    mesh = plsc.VectorSubcoreMesh(core_axis_name="c", subcore_axis_name="s")
    row_t = jax.ShapeDtypeStruct((B, D), jnp.float32)

    @functools.partial(
        pl.kernel,
        out_type=(row_t, row_t, row_t),
        mesh=mesh,
        compiler_params=pltpu.CompilerParams(needs_layout_passes=False),
        scratch_types=[
            pltpu.VMEM((3 * BPW,), jnp.int32),
            pltpu.VMEM((BPW,), jnp.int32),
            pltpu.VMEM((BPW,), jnp.int32),
            pltpu.VMEM((BPW,), jnp.int32),
            pltpu.VMEM((BPW, D), jnp.float32),
            pltpu.VMEM((BPW, D), jnp.float32),
            pltpu.VMEM((BPW, D), jnp.float32),
            pltpu.SemaphoreType.DMA,
        ],
    )
    def body(sample_flat, entity, relation, out_h, out_r, out_t,
             sblk, ih_v, ir_v, it_v, buf_h, buf_r, buf_t, sem):
        wid = lax.axis_index("s") * 2 + lax.axis_index("c")
        b0 = wid * BPW
        lanes = lax.iota(jnp.int32, 16)

        pltpu.sync_copy(sample_flat.at[pl.ds(3 * b0, 3 * BPW)], sblk)
        for m in range(BPW // 16):
            row3 = (m * 16 + lanes) * 3
            for c, dst in ((0, ih_v), (1, ir_v), (2, it_v)):
                dst[pl.ds(m * 16, 16)] = plsc.load_gather(sblk, [row3 + c])

        ch = pltpu.async_copy(entity.at[ih_v], buf_h, sem)
        cr = pltpu.async_copy(relation.at[ir_v], buf_r, sem)
        ct = pltpu.async_copy(entity.at[it_v], buf_t, sem)
        ch.wait()
        pltpu.sync_copy(buf_h, out_h.at[pl.ds(b0, BPW)])
        cr.wait()
        pltpu.sync_copy(buf_r, out_r.at[pl.ds(b0, BPW)])
        ct.wait()
        pltpu.sync_copy(buf_t, out_t.at[pl.ds(b0, BPW)])

    return body


_sc_gather = _make_sc_gather()


def _tc_norm_body(h_ref, r_ref, t_ref, o_ref):
    def nrm(x):
        s = jnp.sum(x * x, axis=-1, keepdims=True)
        return x / jnp.maximum(jnp.sqrt(s), 1e-12)

    o_ref[...] = jnp.concatenate(
        [nrm(h_ref[...])[:, None, :],
         nrm(r_ref[...])[:, None, :],
         nrm(t_ref[...])[:, None, :]], axis=1)


_tc_norm = pl.pallas_call(
    _tc_norm_body,
    grid=(B // BT,),
    in_specs=[pl.BlockSpec((BT, D), lambda i: (i, 0))] * 3,
    out_specs=pl.BlockSpec((BT, 3, D), lambda i: (i, 0, 0)),
    out_shape=jax.ShapeDtypeStruct((B, 3, D), jnp.float32),
)


def kernel(sample, entity_emb, relation_emb, loss_emb):
    del loss_emb
    h, r, t = _sc_gather(sample.reshape(-1).astype(jnp.int32),
                         entity_emb, relation_emb)
    return _tc_norm(h, r, t)


# speedup vs baseline: 1.9278x; 1.1046x over previous
import functools

import jax
import jax.numpy as jnp
from jax import lax
from jax.experimental import pallas as pl
from jax.experimental.pallas import tpu as pltpu
from jax.experimental.pallas import tpu_sc as plsc

ENTITY_N = 100000
RELATION_N = 1000
D = 128
B = 4096
NW = 32
BPW = B // NW
BT = 512


def _make_sc_gather():
    mesh = plsc.VectorSubcoreMesh(core_axis_name="c", subcore_axis_name="s")

    @functools.partial(
        pl.kernel,
        out_type=jax.ShapeDtypeStruct((3, B, D), jnp.float32),
        mesh=mesh,
        compiler_params=pltpu.CompilerParams(needs_layout_passes=False),
        scratch_types=[
            pltpu.VMEM((BPW, 3), jnp.int32),
            pltpu.VMEM((BPW,), jnp.int32),
            pltpu.VMEM((BPW,), jnp.int32),
            pltpu.VMEM((BPW,), jnp.int32),
            pltpu.VMEM((BPW, D), jnp.float32),
            pltpu.VMEM((BPW, D), jnp.float32),
            pltpu.VMEM((BPW, D), jnp.float32),
            pltpu.SemaphoreType.DMA,
        ],
    )
    def body(sample, entity, relation, out,
             sblk, ih_v, ir_v, it_v, buf_h, buf_r, buf_t, sem):
        wid = lax.axis_index("s") * 2 + lax.axis_index("c")
        b0 = wid * BPW
        lanes = lax.iota(jnp.int32, 16)

        pltpu.sync_copy(sample.at[pl.ds(b0, BPW)], sblk)
        for m in range(BPW // 16):
            rows = m * 16 + lanes
            for c, dst in ((0, ih_v), (1, ir_v), (2, it_v)):
                col = jnp.full((16,), c, jnp.int32)
                dst[pl.ds(m * 16, 16)] = plsc.load_gather(sblk, [rows, col])

        ch = pltpu.async_copy(entity.at[ih_v], buf_h, sem)
        cr = pltpu.async_copy(relation.at[ir_v], buf_r, sem)
        ct = pltpu.async_copy(entity.at[it_v], buf_t, sem)
        ch.wait()
        pltpu.sync_copy(buf_h, out.at[0, pl.ds(b0, BPW)])
        cr.wait()
        pltpu.sync_copy(buf_r, out.at[1, pl.ds(b0, BPW)])
        ct.wait()
        pltpu.sync_copy(buf_t, out.at[2, pl.ds(b0, BPW)])

    return body


_sc_gather = _make_sc_gather()


def _tc_norm_body(x_ref, o_ref):
    x = x_ref[0]
    s = jnp.sum(x * x, axis=-1, keepdims=True)
    o_ref[0] = x / jnp.maximum(jnp.sqrt(s), 1e-12)


_tc_norm = pl.pallas_call(
    _tc_norm_body,
    grid=(3, B // BT),
    in_specs=[pl.BlockSpec((1, BT, D), lambda c, i: (c, i, 0))],
    out_specs=pl.BlockSpec((1, BT, D), lambda c, i: (c, i, 0)),
    out_shape=jax.ShapeDtypeStruct((3, B, D), jnp.float32),
)


def kernel(sample, entity_emb, relation_emb, loss_emb):
    del loss_emb
    g = _sc_gather(sample.astype(jnp.int32), entity_emb, relation_emb)
    return _tc_norm(g).transpose(1, 0, 2)
